# baseline (speedup 1.0000x reference)
import functools

import jax
import jax.numpy as jnp
import numpy as np
from jax.experimental import pallas as pl
from jax.experimental.pallas import tpu as pltpu

_BINS = 10
_EDGES = [np.float32(np.float64(i) / _BINS) for i in range(_BINS + 1)]
_EDGES[-1] = np.float32(1.0 + 1e-06)
_CHUNK = 8
_BIG = np.float32(8192.0)
_INV_BIG = np.float32(1.0 / 8192.0)

_DW = np.zeros((256, 128), np.float32)
_TW = np.zeros((256, 256), np.float32)
for _k in range(128):
    _DW[2 * _k, _k] = 1.0
    _DW[2 * _k + 1, _k] = -1.0
    _TW[2 * _k, _k] = 1.0
    _TW[2 * _k + 1, 128 + _k] = 1.0

_DOT_DIMS = (((1,), (0,)), ((), ()))


def _split_dot(a, w_ref):
    hi = a.astype(jnp.bfloat16)
    lo = (a - hi.astype(jnp.float32)).astype(jnp.bfloat16)
    w = w_ref[...]
    acc = jax.lax.dot_general(hi, w, _DOT_DIMS,
                              preferred_element_type=jnp.float32)
    return acc + jax.lax.dot_general(lo, w, _DOT_DIMS,
                                     preferred_element_type=jnp.float32)


def _ghmc_kernel(x_ref, t_ref, dw_ref, tw_ref, o_ref, acc_ref, d_buf, tt_buf,
                 *, n_rows, n_steps, tot):
    step = pl.program_id(0)

    @pl.when(step == 0)
    def _init():
        acc_ref[...] = jnp.zeros_like(acc_ref)

    d_buf[...] = _split_dot(x_ref[...], dw_ref)
    tt_buf[...] = _split_dot(t_ref[...], tw_ref)

    n_chunks = n_rows // _CHUNK

    carry = [jnp.zeros((_CHUNK, 128), jnp.float32) for _ in range(_BINS + 1)]
    for i in range(n_chunks):
        d = d_buf[pl.ds(i * _CHUNK, _CHUNK), :]
        t0 = tt_buf[pl.ds(i * _CHUNK, _CHUNK), 0:128]
        t1 = tt_buf[pl.ds(i * _CHUNK, _CHUNK), 128:256]

        ad = jnp.abs(d)
        q = jnp.exp(-ad)
        u = 1.0 + q
        r = 1.0 / u
        p0 = jnp.where(d >= 0.0, r, 1.0 - r)
        p1 = 1.0 - p0
        g = jnp.abs(p0 - t0) * t0 + jnp.abs(p1 - t1) * t1
        z = jnp.where(t0 >= t1, -d, d)
        ce = jnp.maximum(z, 0.0) + jnp.log(u)
        w = ce + _BIG

        for j in range(_BINS):
            m = g >= _EDGES[j + 1]
            carry[j] = carry[j] + jnp.where(m, w, 0.0)
        carry[_BINS] = carry[_BINS] + ce

    for j in range(_BINS):
        cnt = jnp.floor(carry[j] * _INV_BIG)
        s = carry[j] - cnt * _BIG
        acc_ref[j] += cnt
        acc_ref[_BINS + 1 + j] += s
    acc_ref[_BINS] += carry[_BINS]

    @pl.when(step == n_steps - 1)
    def _fin():
        C = [jnp.sum(acc_ref[j]) for j in range(_BINS)]
        S = [jnp.sum(acc_ref[_BINS + j]) for j in range(_BINS + 1)]
        cnts = [tot - C[0]]
        sums = [S[0] - S[1]]
        for b in range(1, _BINS):
            cnts.append(C[b - 1] - C[b])
            sums.append(S[b] - S[b + 1])
        n = jnp.zeros((), jnp.float32)
        total = jnp.zeros((), jnp.float32)
        for b in range(_BINS):
            nonempty = cnts[b] > 0.0
            n = n + nonempty.astype(jnp.float32)
            contrib = sums[b] / (np.float32(0.1) * jnp.maximum(cnts[b], 1.0))
            total = total + jnp.where(nonempty, contrib, 0.0)
        o_ref[0, 0] = total / jnp.maximum(n, 1.0)


def kernel(input, target):
    n, c = input.shape
    assert c == 2
    m = n // 128
    x = input.reshape(m, 256)
    t = target.reshape(m, 256)
    rows = min(512, m)
    steps = m // rows
    out = pl.pallas_call(
        functools.partial(
            _ghmc_kernel, n_rows=rows, n_steps=steps, tot=np.float32(n)
        ),
        grid=(steps,),
        in_specs=[
            pl.BlockSpec((rows, 256), lambda i: (i, 0)),
            pl.BlockSpec((rows, 256), lambda i: (i, 0)),
            pl.BlockSpec((256, 128), lambda i: (0, 0)),
            pl.BlockSpec((256, 256), lambda i: (0, 0)),
        ],
        out_specs=pl.BlockSpec(memory_space=pltpu.SMEM),
        out_shape=jax.ShapeDtypeStruct((1, 1), jnp.float32),
        scratch_shapes=[
            pltpu.VMEM((2 * _BINS + 1, 8, 128), jnp.float32),
            pltpu.VMEM((rows, 128), jnp.float32),
            pltpu.VMEM((rows, 256), jnp.float32),
        ],
    )(x, t, jnp.asarray(_DW, jnp.bfloat16), jnp.asarray(_TW, jnp.bfloat16))
    return out[0, 0]
